# baseline (speedup 1.0000x reference)
import dataclasses
import functools
import jax
import jax.numpy as jnp
from jax import lax
from jax.experimental import pallas as pl
from jax.experimental.pallas import tpu as pltpu
from jax.experimental.pallas import tpu_sc as plsc

E = 320000
D = 128
S = 256
BES = 80
NBS = E // BES
L = 16


def _sc_kernel_body(x_hbm, seg_hbm, w_hbm, b_hbm,
                    hout_hbm, wout_hbm,
                    acc_ref, wp_ref, bv_ref, wbuf_ref, idx_ref,
                    wsplat_ref):
    c = lax.axis_index("c")
    s = lax.axis_index("s")
    pltpu.sync_copy(w_hbm, wp_ref)
    pltpu.sync_copy(b_hbm, bv_ref)

    zero16 = jnp.zeros((L,), jnp.float32)

    @pl.loop(0, S)
    def _(r):
        for j in range(D // L):
            acc_ref[r, pl.ds(j * L, L)] = zero16

    @pl.loop(0, S, step=L)
    def _(k):
        idx_ref[pl.ds(k, L)] = jnp.arange(L, dtype=jnp.int32) + k

    bv = bv_ref[...]
    zvec = jnp.zeros((L,), jnp.float32)
    zidx = jnp.zeros((L,), jnp.int32)

    @pl.loop(0, D)
    def _(d):
        wsplat_ref[d, pl.ds(0, L)] = plsc.load_gather(wp_ref, [zidx + d])

    NG = BES // L

    def blk(x_v, seg_v, wout_v):
        rowis = [jnp.arange(L, dtype=jnp.int32) + (g * L) for g in range(NG)]
        accs = [zvec] * NG
        civ = zidx
        for d in range(D):
            wv = wsplat_ref[d, pl.ds(0, L)]
            for g in range(NG):
                accs[g] = accs[g] + plsc.load_gather(x_v, [rowis[g], civ]) * wv
            civ = civ + 1

        wvecs = []
        for g in range(NG):
            wvg = 1.0 / (1.0 + jnp.exp(-(accs[g] + bv)))
            wout_v[0, pl.ds(g * L, L)] = wvg
            wvecs.append(wvg)

        for gi in range(NG):
            g = gi * L
            sv = seg_v[0, pl.ds(g, L)]
            wvv = wvecs[gi]
            s_first = sv[0]
            s_last = sv[L - 1]
            ws = [lax.broadcast(wvv[u], (L,)) for u in range(L)]

            @pl.when(s_first == s_last)
            def _():
                for j in range(D // L):
                    ts = [x_v[g + q, pl.ds(j * L, L)] * ws[q]
                          for q in range(4)]
                    for u in range(4, L):
                        ts[u % 4] = ts[u % 4] + \
                            x_v[g + u, pl.ds(j * L, L)] * ws[u]
                    plsc.addupdate(acc_ref.at[s_first, pl.ds(j * L, L)],
                                   (ts[0] + ts[1]) + (ts[2] + ts[3]))

            @pl.when(s_first != s_last)
            def _():
                for u in range(L):
                    se = sv[u]
                    for j in range(D // L):
                        xv = x_v[g + u, pl.ds(j * L, L)]
                        plsc.addupdate(acc_ref.at[se, pl.ds(j * L, L)],
                                       xv * ws[u])

    pltpu.emit_pipeline(
        blk,
        grid=(NBS,),
        in_specs=[
            pl.BlockSpec((BES, D), lambda i: (i, 0)),
            pl.BlockSpec((1, BES), lambda i: (i, 0)),
        ],
        out_specs=[
            pl.BlockSpec((1, BES), lambda i: (i, 0)),
        ],
        core_axis_name=("c", "s"),
        dimension_semantics=(pltpu.PARALLEL,),
    )(x_hbm, seg_hbm, wout_hbm)

    wid = c * 16 + s
    pltpu.sync_copy(acc_ref, hout_hbm.at[wid])


def kernel(edge_feats, segment_ids, W, b):
    seg2 = segment_ids.astype(jnp.int32).reshape(NBS, BES)
    wflat = W.astype(jnp.float32).reshape(D)
    b16 = jnp.broadcast_to(b.astype(jnp.float32), (L,))
    mesh = plsc.VectorSubcoreMesh(core_axis_name="c", subcore_axis_name="s")
    cp = pltpu.CompilerParams()
    if "needs_layout_passes" in pltpu.CompilerParams.__dataclass_fields__:
        cp = dataclasses.replace(cp, needs_layout_passes=False)
    sc = functools.partial(
        pl.kernel,
        mesh=mesh,
        compiler_params=cp,
        out_type=[
            jax.ShapeDtypeStruct((32, S, D), jnp.float32),
            jax.ShapeDtypeStruct((NBS, BES), jnp.float32),
        ],
        scratch_types=[
            pltpu.VMEM((S, D), jnp.float32),
            pltpu.VMEM((D,), jnp.float32),
            pltpu.VMEM((L,), jnp.float32),
            pltpu.VMEM((BES,), jnp.float32),
            pltpu.VMEM((S,), jnp.int32),
            pltpu.VMEM((D, L), jnp.float32),
        ],
    )(_sc_kernel_body)
    hparts, wout = sc(edge_feats, seg2, wflat, b16)
    h = hparts.sum(axis=0)
    return (h, wout.reshape(E, 1))


# speedup vs baseline: 8.6245x; 8.6245x over previous
import dataclasses
import functools
import jax
import jax.numpy as jnp
from jax import lax
from jax.experimental import pallas as pl
from jax.experimental.pallas import tpu as pltpu
from jax.experimental.pallas import tpu_sc as plsc

E = 320000
D = 128
S = 256

BE = 7232
NB = 40
E_TC = BE * NB
WS = 32

L = 16
BES = 64
NG = BES // L
E_SC = E - E_TC
NBS = E_SC // BES
OFF = E_TC // BES


def _tc_body(meta_ref, seg_ref, x_ref, w_ref, b_ref, iota_ref,
             out_ref, wout_ref):
    i = pl.program_id(0)
    x = x_ref[...]
    xb = x.astype(jnp.bfloat16)
    w1 = w_ref[...].astype(jnp.bfloat16)
    logits_t = jax.lax.dot_general(
        w1, xb, (((1,), (1,)), ((), ())),
        preferred_element_type=jnp.float32)
    w_t = jax.nn.sigmoid(logits_t + b_ref[0, 0])
    wout_ref[...] = w_t.reshape(1, 1, BE)

    @pl.when(i == 0)
    def _():
        out_ref[...] = jnp.zeros_like(out_ref)

    seg = seg_ref[0]
    w_tb = w_t.astype(jnp.bfloat16)
    first = meta_ref[0, 0, 0]
    last = meta_ref[0, 0, 1]
    for jw in range(S // WS):
        base = jw * WS

        @pl.when((first < base + WS) & (last >= base))
        def _():
            onehot_w = jnp.where(iota_ref[base:base + WS, :] == seg,
                                 w_tb, jnp.bfloat16(0))
            out_ref[base:base + WS, :] += jax.lax.dot_general(
                onehot_w, xb, (((1,), (0,)), ((), ())),
                preferred_element_type=jnp.float32)


def _tc_part(edge_feats, segi, W, b):
    seg_tc = segi[:E_TC]
    meta = jnp.stack([seg_tc[::BE], seg_tc[BE - 1::BE]],
                     axis=1).reshape(NB, 1, 2)
    seg3 = seg_tc.astype(jnp.bfloat16).reshape(NB, 1, BE)
    b2 = b.astype(jnp.float32).reshape(1, 1)
    w1 = W.astype(jnp.float32).reshape(1, D)
    iota = jnp.arange(S, dtype=jnp.bfloat16).reshape(S, 1)
    return pl.pallas_call(
        _tc_body,
        grid=(NB,),
        in_specs=[
            pl.BlockSpec((1, 1, 2), lambda i: (i, 0, 0),
                         memory_space=pltpu.SMEM),
            pl.BlockSpec((1, 1, BE), lambda i: (i, 0, 0)),
            pl.BlockSpec((BE, D), lambda i: (i, 0)),
            pl.BlockSpec((1, D), lambda i: (0, 0)),
            pl.BlockSpec((1, 1), lambda i: (0, 0)),
            pl.BlockSpec((S, 1), lambda i: (0, 0)),
        ],
        out_specs=[
            pl.BlockSpec((S, D), lambda i: (0, 0)),
            pl.BlockSpec((1, 1, BE), lambda i: (i, 0, 0)),
        ],
        out_shape=[
            jax.ShapeDtypeStruct((S, D), jnp.float32),
            jax.ShapeDtypeStruct((NB, 1, BE), jnp.float32),
        ],
        compiler_params=pltpu.CompilerParams(
            dimension_semantics=("arbitrary",),
        ),
    )(meta, seg3, edge_feats, w1, b2, iota)


def _sc_kernel_body(x_hbm, seg_hbm, w_hbm, b_hbm,
                    hout_hbm, wout_hbm,
                    acc_ref, wp_ref, bv_ref, wsplat_ref):
    c = lax.axis_index("c")
    s = lax.axis_index("s")
    pltpu.sync_copy(w_hbm, wp_ref)
    pltpu.sync_copy(b_hbm, bv_ref)

    zero16 = jnp.zeros((L,), jnp.float32)

    @pl.loop(0, S)
    def _(r):
        for j in range(D // L):
            acc_ref[r, pl.ds(j * L, L)] = zero16

    bv = bv_ref[...]
    zvec = jnp.zeros((L,), jnp.float32)
    zidx = jnp.zeros((L,), jnp.int32)

    @pl.loop(0, D)
    def _(d):
        wsplat_ref[d, pl.ds(0, L)] = plsc.load_gather(wp_ref, [zidx + d])

    def blk(x_v, seg_v, wout_v):
        rowis = [jnp.arange(L, dtype=jnp.int32) + (g * L) for g in range(NG)]

        def dbody(i, carry):
            accs, ci = carry[:NG], carry[NG]
            d = 4 * i
            news = list(accs)
            for q in range(4):
                wv = wsplat_ref[d + q, pl.ds(0, L)]
                for g in range(NG):
                    gv = plsc.load_gather(x_v, [rowis[g], ci + q])
                    news[g] = news[g] + gv * wv
            return (*news, ci + 4)

        out = lax.fori_loop(0, D // 4, dbody,
                            tuple(zvec for _ in range(NG)) + (zidx,))
        wvecs = []
        for g in range(NG):
            wvg = 1.0 / (1.0 + jnp.exp(-(out[g] + bv)))
            wout_v[0, pl.ds(g * L, L)] = wvg
            wvecs.append(wvg)

        for gi in range(NG):
            g = gi * L
            sv = seg_v[0, pl.ds(g, L)]
            wvv = wvecs[gi]
            s_first = sv[0]
            s_last = sv[L - 1]
            ws = [lax.broadcast(wvv[u], (L,)) for u in range(L)]

            @pl.when(s_first == s_last)
            def _():
                for j in range(D // L):
                    ts = [x_v[g + q, pl.ds(j * L, L)] * ws[q]
                          for q in range(4)]
                    for u in range(4, L):
                        ts[u % 4] = ts[u % 4] + \
                            x_v[g + u, pl.ds(j * L, L)] * ws[u]
                    plsc.addupdate(acc_ref.at[s_first, pl.ds(j * L, L)],
                                   (ts[0] + ts[1]) + (ts[2] + ts[3]))

            @pl.when(s_first != s_last)
            def _():
                for u in range(L):
                    se = sv[u]
                    for j in range(D // L):
                        xv = x_v[g + u, pl.ds(j * L, L)]
                        plsc.addupdate(acc_ref.at[se, pl.ds(j * L, L)],
                                       xv * ws[u])

    pltpu.emit_pipeline(
        blk,
        grid=(NBS,),
        in_specs=[
            pl.BlockSpec((BES, D), lambda i: (i + OFF, 0)),
            pl.BlockSpec((1, BES), lambda i: (i, 0)),
        ],
        out_specs=[
            pl.BlockSpec((1, BES), lambda i: (i, 0)),
        ],
        core_axis_name=("c", "s"),
        dimension_semantics=(pltpu.PARALLEL,),
    )(x_hbm, seg_hbm, wout_hbm)

    wid = c * 16 + s
    pltpu.sync_copy(acc_ref, hout_hbm.at[wid])


def _sc_part(edge_feats, segi, W, b):
    seg2 = segi[E_TC:].reshape(NBS, BES)
    wflat = W.astype(jnp.float32).reshape(D)
    b16 = jnp.broadcast_to(b.astype(jnp.float32), (L,))
    mesh = plsc.VectorSubcoreMesh(core_axis_name="c", subcore_axis_name="s")
    cp = pltpu.CompilerParams()
    if "needs_layout_passes" in pltpu.CompilerParams.__dataclass_fields__:
        cp = dataclasses.replace(cp, needs_layout_passes=False)
    sc = functools.partial(
        pl.kernel,
        mesh=mesh,
        compiler_params=cp,
        out_type=[
            jax.ShapeDtypeStruct((32, S, D), jnp.float32),
            jax.ShapeDtypeStruct((NBS, BES), jnp.float32),
        ],
        scratch_types=[
            pltpu.VMEM((S, D), jnp.float32),
            pltpu.VMEM((D,), jnp.float32),
            pltpu.VMEM((L,), jnp.float32),
            pltpu.VMEM((D, L), jnp.float32),
        ],
    )(_sc_kernel_body)
    return sc(edge_feats, seg2, wflat, b16)


def kernel(edge_feats, segment_ids, W, b):
    segi = segment_ids.astype(jnp.int32)
    h_tc, w_tc = _tc_part(edge_feats, segi, W, b)
    h_sc, w_sc = _sc_part(edge_feats, segi, W, b)
    h = h_tc + h_sc.sum(axis=0)
    weights = jnp.concatenate(
        [w_tc.reshape(E_TC), w_sc.reshape(E_SC)]).reshape(E, 1)
    return (h, weights)


# speedup vs baseline: 9.3685x; 1.0863x over previous
import dataclasses
import functools
import jax
import jax.numpy as jnp
from jax import lax
from jax.experimental import pallas as pl
from jax.experimental.pallas import tpu as pltpu
from jax.experimental.pallas import tpu_sc as plsc

E = 320000
D = 128
S = 256

BE = 7616
NB = 40
E_TC = BE * NB
WS = 32

L = 16
BES = 64
NG = BES // L
E_SC = E - E_TC
NBS = E_SC // BES
OFF = E_TC // BES


def _tc_body(meta_ref, seg_ref, x_ref, w_ref, b_ref, iota_ref,
             out_ref, wout_ref):
    i = pl.program_id(0)
    x = x_ref[...]
    xb = x.astype(jnp.bfloat16)
    w1 = w_ref[...].astype(jnp.bfloat16)
    logits_t = jax.lax.dot_general(
        w1, xb, (((1,), (1,)), ((), ())),
        preferred_element_type=jnp.float32)
    w_t = jax.nn.sigmoid(logits_t + b_ref[0, 0])
    wout_ref[...] = w_t.reshape(1, 1, BE)

    @pl.when(i == 0)
    def _():
        out_ref[...] = jnp.zeros_like(out_ref)

    seg = seg_ref[0]
    w_tb = w_t.astype(jnp.bfloat16)
    first = meta_ref[0, 0, 0]
    last = meta_ref[0, 0, 1]
    for jw in range(S // WS):
        base = jw * WS

        @pl.when((first < base + WS) & (last >= base))
        def _():
            onehot_w = jnp.where(iota_ref[base:base + WS, :] == seg,
                                 w_tb, jnp.bfloat16(0))
            out_ref[base:base + WS, :] += jax.lax.dot_general(
                onehot_w, xb, (((1,), (0,)), ((), ())),
                preferred_element_type=jnp.float32)


def _tc_part(edge_feats, segi, W, b):
    seg_tc = segi[:E_TC]
    meta = jnp.stack([seg_tc[::BE], seg_tc[BE - 1::BE]],
                     axis=1).reshape(NB, 1, 2)
    seg3 = seg_tc.astype(jnp.bfloat16).reshape(NB, 1, BE)
    b2 = b.astype(jnp.float32).reshape(1, 1)
    w1 = W.astype(jnp.float32).reshape(1, D)
    iota = jnp.arange(S, dtype=jnp.bfloat16).reshape(S, 1)
    return pl.pallas_call(
        _tc_body,
        grid=(NB,),
        in_specs=[
            pl.BlockSpec((1, 1, 2), lambda i: (i, 0, 0),
                         memory_space=pltpu.SMEM),
            pl.BlockSpec((1, 1, BE), lambda i: (i, 0, 0)),
            pl.BlockSpec((BE, D), lambda i: (i, 0)),
            pl.BlockSpec((1, D), lambda i: (0, 0)),
            pl.BlockSpec((1, 1), lambda i: (0, 0)),
            pl.BlockSpec((S, 1), lambda i: (0, 0)),
        ],
        out_specs=[
            pl.BlockSpec((S, D), lambda i: (0, 0)),
            pl.BlockSpec((1, 1, BE), lambda i: (i, 0, 0)),
        ],
        out_shape=[
            jax.ShapeDtypeStruct((S, D), jnp.float32),
            jax.ShapeDtypeStruct((NB, 1, BE), jnp.float32),
        ],
        compiler_params=pltpu.CompilerParams(
            dimension_semantics=("arbitrary",),
        ),
    )(meta, seg3, edge_feats, w1, b2, iota)


def _sc_kernel_body(x_hbm, seg_hbm, w_hbm, b_hbm,
                    hout_hbm, wout_hbm,
                    acc_ref, wp_ref, bv_ref, wsplat_ref):
    c = lax.axis_index("c")
    s = lax.axis_index("s")
    pltpu.sync_copy(w_hbm, wp_ref)
    pltpu.sync_copy(b_hbm, bv_ref)

    zero16 = jnp.zeros((L,), jnp.float32)

    @pl.loop(0, S)
    def _(r):
        for j in range(D // L):
            acc_ref[r, pl.ds(j * L, L)] = zero16

    bv = bv_ref[...]
    zvec = jnp.zeros((L,), jnp.float32)
    zidx = jnp.zeros((L,), jnp.int32)

    @pl.loop(0, D)
    def _(d):
        wsplat_ref[d, pl.ds(0, L)] = plsc.load_gather(wp_ref, [zidx + d])

    def blk(x_v, seg_v, wout_v):
        rowis = [jnp.arange(L, dtype=jnp.int32) + (g * L) for g in range(NG)]

        def dbody(i, carry):
            accs, ci = carry[:NG], carry[NG]
            d = 4 * i
            news = list(accs)
            for q in range(4):
                wv = wsplat_ref[d + q, pl.ds(0, L)]
                for g in range(NG):
                    gv = plsc.load_gather(x_v, [rowis[g], ci + q])
                    news[g] = news[g] + gv * wv
            return (*news, ci + 4)

        out = lax.fori_loop(0, D // 4, dbody,
                            tuple(zvec for _ in range(NG)) + (zidx,))
        wvecs = []
        for g in range(NG):
            wvg = 1.0 / (1.0 + jnp.exp(-(out[g] + bv)))
            wout_v[0, pl.ds(g * L, L)] = wvg
            wvecs.append(wvg)

        for gi in range(NG):
            g = gi * L
            sv = seg_v[0, pl.ds(g, L)]
            wvv = wvecs[gi]
            s_first = sv[0]
            s_last = sv[L - 1]
            ws = [lax.broadcast(wvv[u], (L,)) for u in range(L)]

            @pl.when(s_first == s_last)
            def _():
                for j in range(D // L):
                    ts = [x_v[g + q, pl.ds(j * L, L)] * ws[q]
                          for q in range(4)]
                    for u in range(4, L):
                        ts[u % 4] = ts[u % 4] + \
                            x_v[g + u, pl.ds(j * L, L)] * ws[u]
                    plsc.addupdate(acc_ref.at[s_first, pl.ds(j * L, L)],
                                   (ts[0] + ts[1]) + (ts[2] + ts[3]))

            @pl.when(s_first != s_last)
            def _():
                for u in range(L):
                    se = sv[u]
                    for j in range(D // L):
                        xv = x_v[g + u, pl.ds(j * L, L)]
                        plsc.addupdate(acc_ref.at[se, pl.ds(j * L, L)],
                                       xv * ws[u])

    pltpu.emit_pipeline(
        blk,
        grid=(NBS,),
        in_specs=[
            pl.BlockSpec((BES, D), lambda i: (i + OFF, 0)),
            pl.BlockSpec((1, BES), lambda i: (i, 0)),
        ],
        out_specs=[
            pl.BlockSpec((1, BES), lambda i: (i, 0)),
        ],
        core_axis_name=("c", "s"),
        dimension_semantics=(pltpu.PARALLEL,),
    )(x_hbm, seg_hbm, wout_hbm)

    wid = c * 16 + s
    pltpu.sync_copy(acc_ref, hout_hbm.at[wid])


def _sc_part(edge_feats, segi, W, b):
    seg2 = segi[E_TC:].reshape(NBS, BES)
    wflat = W.astype(jnp.float32).reshape(D)
    b16 = jnp.broadcast_to(b.astype(jnp.float32), (L,))
    mesh = plsc.VectorSubcoreMesh(core_axis_name="c", subcore_axis_name="s")
    cp = pltpu.CompilerParams()
    if "needs_layout_passes" in pltpu.CompilerParams.__dataclass_fields__:
        cp = dataclasses.replace(cp, needs_layout_passes=False)
    sc = functools.partial(
        pl.kernel,
        mesh=mesh,
        compiler_params=cp,
        out_type=[
            jax.ShapeDtypeStruct((32, S, D), jnp.float32),
            jax.ShapeDtypeStruct((NBS, BES), jnp.float32),
        ],
        scratch_types=[
            pltpu.VMEM((S, D), jnp.float32),
            pltpu.VMEM((D,), jnp.float32),
            pltpu.VMEM((L,), jnp.float32),
            pltpu.VMEM((D, L), jnp.float32),
        ],
    )(_sc_kernel_body)
    return sc(edge_feats, seg2, wflat, b16)


def kernel(edge_feats, segment_ids, W, b):
    segi = segment_ids.astype(jnp.int32)
    h_tc, w_tc = _tc_part(edge_feats, segi, W, b)
    h_sc, w_sc = _sc_part(edge_feats, segi, W, b)
    h = h_tc + h_sc.sum(axis=0)
    weights = jnp.concatenate(
        [w_tc.reshape(E_TC), w_sc.reshape(E_SC)]).reshape(E, 1)
    return (h, weights)


# speedup vs baseline: 9.5377x; 1.0181x over previous
import dataclasses
import functools
import jax
import jax.numpy as jnp
from jax import lax
from jax.experimental import pallas as pl
from jax.experimental.pallas import tpu as pltpu
from jax.experimental.pallas import tpu_sc as plsc

E = 320000
D = 128
S = 256

BE = 7808
NB = 40
E_TC = BE * NB
WS = 32

L = 16
BES = 64
NG = BES // L
E_SC = E - E_TC
NBS = E_SC // BES
OFF = E_TC // BES


def _tc_body(meta_ref, seg_ref, x_ref, w_ref, b_ref, iota_ref,
             out_ref, wout_ref):
    i = pl.program_id(0)
    x = x_ref[...]
    xb = x.astype(jnp.bfloat16)
    w1 = w_ref[...].astype(jnp.bfloat16)
    logits_t = jax.lax.dot_general(
        w1, xb, (((1,), (1,)), ((), ())),
        preferred_element_type=jnp.float32)
    w_t = jax.nn.sigmoid(logits_t + b_ref[0, 0])
    wout_ref[...] = w_t.reshape(1, 1, BE)

    @pl.when(i == 0)
    def _():
        out_ref[...] = jnp.zeros_like(out_ref)

    seg = seg_ref[0]
    w_tb = w_t.astype(jnp.bfloat16)
    first = meta_ref[0, 0, 0]
    last = meta_ref[0, 0, 1]
    for jw in range(S // WS):
        base = jw * WS

        @pl.when((first < base + WS) & (last >= base))
        def _():
            onehot_w = jnp.where(iota_ref[base:base + WS, :] == seg,
                                 w_tb, jnp.bfloat16(0))
            out_ref[base:base + WS, :] += jax.lax.dot_general(
                onehot_w, xb, (((1,), (0,)), ((), ())),
                preferred_element_type=jnp.float32)


def _tc_part(edge_feats, segi, W, b):
    seg_tc = segi[:E_TC]
    meta = jnp.stack([seg_tc[::BE], seg_tc[BE - 1::BE]],
                     axis=1).reshape(NB, 1, 2)
    seg3 = seg_tc.astype(jnp.bfloat16).reshape(NB, 1, BE)
    b2 = b.astype(jnp.float32).reshape(1, 1)
    w1 = W.astype(jnp.float32).reshape(1, D)
    iota = jnp.arange(S, dtype=jnp.bfloat16).reshape(S, 1)
    return pl.pallas_call(
        _tc_body,
        grid=(NB,),
        in_specs=[
            pl.BlockSpec((1, 1, 2), lambda i: (i, 0, 0),
                         memory_space=pltpu.SMEM),
            pl.BlockSpec((1, 1, BE), lambda i: (i, 0, 0)),
            pl.BlockSpec((BE, D), lambda i: (i, 0)),
            pl.BlockSpec((1, D), lambda i: (0, 0)),
            pl.BlockSpec((1, 1), lambda i: (0, 0)),
            pl.BlockSpec((S, 1), lambda i: (0, 0)),
        ],
        out_specs=[
            pl.BlockSpec((S, D), lambda i: (0, 0)),
            pl.BlockSpec((1, 1, BE), lambda i: (i, 0, 0)),
        ],
        out_shape=[
            jax.ShapeDtypeStruct((S, D), jnp.float32),
            jax.ShapeDtypeStruct((NB, 1, BE), jnp.float32),
        ],
        compiler_params=pltpu.CompilerParams(
            dimension_semantics=("arbitrary",),
        ),
    )(meta, seg3, edge_feats, w1, b2, iota)


def _sc_kernel_body(x_hbm, seg_hbm, w_hbm, b_hbm,
                    hout_hbm, wout_hbm,
                    acc_ref, wp_ref, bv_ref, wsplat_ref):
    c = lax.axis_index("c")
    s = lax.axis_index("s")
    pltpu.sync_copy(w_hbm, wp_ref)
    pltpu.sync_copy(b_hbm, bv_ref)

    zero16 = jnp.zeros((L,), jnp.float32)

    @pl.loop(0, S)
    def _(r):
        for j in range(D // L):
            acc_ref[r, pl.ds(j * L, L)] = zero16

    bv = bv_ref[...]
    zvec = jnp.zeros((L,), jnp.float32)
    zidx = jnp.zeros((L,), jnp.int32)

    @pl.loop(0, D)
    def _(d):
        wsplat_ref[d, pl.ds(0, L)] = plsc.load_gather(wp_ref, [zidx + d])

    def blk(x_v, seg_v, wout_v):
        rowis = [jnp.arange(L, dtype=jnp.int32) + (g * L) for g in range(NG)]

        def dbody(i, carry):
            accs, ci = carry[:NG], carry[NG]
            d = 4 * i
            news = list(accs)
            for q in range(4):
                wv = wsplat_ref[d + q, pl.ds(0, L)]
                for g in range(NG):
                    gv = plsc.load_gather(x_v, [rowis[g], ci + q])
                    news[g] = news[g] + gv * wv
            return (*news, ci + 4)

        out = lax.fori_loop(0, D // 4, dbody,
                            tuple(zvec for _ in range(NG)) + (zidx,))
        wvecs = []
        for g in range(NG):
            wvg = 1.0 / (1.0 + jnp.exp(-(out[g] + bv)))
            wout_v[0, pl.ds(g * L, L)] = wvg
            wvecs.append(wvg)

        for gi in range(NG):
            g = gi * L
            sv = seg_v[0, pl.ds(g, L)]
            wvv = wvecs[gi]
            s_first = sv[0]
            s_last = sv[L - 1]
            ws = [lax.broadcast(wvv[u], (L,)) for u in range(L)]

            @pl.when(s_first == s_last)
            def _():
                for j in range(D // L):
                    ts = [x_v[g + q, pl.ds(j * L, L)] * ws[q]
                          for q in range(4)]
                    for u in range(4, L):
                        ts[u % 4] = ts[u % 4] + \
                            x_v[g + u, pl.ds(j * L, L)] * ws[u]
                    plsc.addupdate(acc_ref.at[s_first, pl.ds(j * L, L)],
                                   (ts[0] + ts[1]) + (ts[2] + ts[3]))

            @pl.when(s_first != s_last)
            def _():
                for u in range(L):
                    se = sv[u]
                    for j in range(D // L):
                        xv = x_v[g + u, pl.ds(j * L, L)]
                        plsc.addupdate(acc_ref.at[se, pl.ds(j * L, L)],
                                       xv * ws[u])

    pltpu.emit_pipeline(
        blk,
        grid=(NBS,),
        in_specs=[
            pl.BlockSpec((BES, D), lambda i: (i + OFF, 0)),
            pl.BlockSpec((1, BES), lambda i: (i, 0)),
        ],
        out_specs=[
            pl.BlockSpec((1, BES), lambda i: (i, 0)),
        ],
        core_axis_name=("c", "s"),
        dimension_semantics=(pltpu.PARALLEL,),
    )(x_hbm, seg_hbm, wout_hbm)

    wid = c * 16 + s
    pltpu.sync_copy(acc_ref, hout_hbm.at[wid])


def _sc_part(edge_feats, segi, W, b):
    seg2 = segi[E_TC:].reshape(NBS, BES)
    wflat = W.astype(jnp.float32).reshape(D)
    b16 = jnp.broadcast_to(b.astype(jnp.float32), (L,))
    mesh = plsc.VectorSubcoreMesh(core_axis_name="c", subcore_axis_name="s")
    cp = pltpu.CompilerParams()
    if "needs_layout_passes" in pltpu.CompilerParams.__dataclass_fields__:
        cp = dataclasses.replace(cp, needs_layout_passes=False)
    sc = functools.partial(
        pl.kernel,
        mesh=mesh,
        compiler_params=cp,
        out_type=[
            jax.ShapeDtypeStruct((32, S, D), jnp.float32),
            jax.ShapeDtypeStruct((NBS, BES), jnp.float32),
        ],
        scratch_types=[
            pltpu.VMEM((S, D), jnp.float32),
            pltpu.VMEM((D,), jnp.float32),
            pltpu.VMEM((L,), jnp.float32),
            pltpu.VMEM((D, L), jnp.float32),
        ],
    )(_sc_kernel_body)
    return sc(edge_feats, seg2, wflat, b16)


def kernel(edge_feats, segment_ids, W, b):
    segi = segment_ids.astype(jnp.int32)
    h_tc, w_tc = _tc_part(edge_feats, segi, W, b)
    h_sc, w_sc = _sc_part(edge_feats, segi, W, b)
    h = h_tc + h_sc.sum(axis=0)
    weights = jnp.concatenate(
        [w_tc.reshape(E_TC), w_sc.reshape(E_SC)]).reshape(E, 1)
    return (h, weights)
